# Initial kernel scaffold; baseline (speedup 1.0000x reference)
import jax
import jax.numpy as jnp
from jax.experimental import pallas as pl


def kernel(node_features, node_attrs, edge_src, edge_dst, edge_attr, edge_embedding, W1, mlp_w0, mlp_w1, W2, W_sc):
    raise NotImplementedError("write your pallas kernel here")


# speedup vs baseline: 2.4114x; 2.4114x over previous
import functools

import numpy as np
import jax
import jax.numpy as jnp
from jax import lax
from jax.experimental import pallas as pl
from jax.experimental.pallas import tpu as pltpu
from jax.experimental.pallas import tpu_sc as plsc

_N = 10000
_E = 320000
_F = 128
_A = 4
_EMB = 16
_HID = 64

_NC = 2
_NS = 16
_NW = _NC * _NS
_EPW = _E // _NW
_CHUNK = 80
_NCHUNK = _EPW // _CHUNK
_NPAD = 10240
_RPT = _NPAD // _NS
_WCH = 128
_NWCH = _RPT // _WCH


def _node_body(nf_ref, na_ref, w1_ref, wsc_ref, x_ref, sc_ref):
    nf = nf_ref[...]
    x_ref[...] = jnp.dot(nf, w1_ref[...], preferred_element_type=jnp.float32)
    acc = jnp.dot(nf, wsc_ref[0], preferred_element_type=jnp.float32) * na_ref[:, 0:1]
    for v in range(1, _A):
        acc = acc + jnp.dot(nf, wsc_ref[v], preferred_element_type=jnp.float32) * na_ref[:, v:v + 1]
    sc_ref[...] = acc


def _node_tc(nf, na, w1s, wsc_t):
    blk = 1000
    grid = _N // blk
    return pl.pallas_call(
        _node_body,
        grid=(grid,),
        in_specs=[
            pl.BlockSpec((blk, _F), lambda i: (i, 0)),
            pl.BlockSpec((blk, _A), lambda i: (i, 0)),
            pl.BlockSpec((_F, _F), lambda i: (0, 0)),
            pl.BlockSpec((_A, _F, _F), lambda i: (0, 0, 0)),
        ],
        out_specs=[
            pl.BlockSpec((blk, _F), lambda i: (i, 0)),
            pl.BlockSpec((blk, _F), lambda i: (i, 0)),
        ],
        out_shape=[
            jax.ShapeDtypeStruct((_N, _F), jnp.float32),
            jax.ShapeDtypeStruct((_N, _F), jnp.float32),
        ],
    )(nf, na, w1s, wsc_t)


def _edge_body(ee_ref, ea_ref, w0_ref, w1_ref, ew_ref):
    h = jnp.dot(ee_ref[...], w0_ref[...], preferred_element_type=jnp.float32)
    h = h * jax.nn.sigmoid(h)
    w = jnp.dot(h, w1_ref[...], preferred_element_type=jnp.float32)
    ew_ref[...] = w * ea_ref[...]


def _edge_tc(ee, ea, w0, w1):
    blk = 6400
    grid = _E // blk
    return pl.pallas_call(
        _edge_body,
        grid=(grid,),
        in_specs=[
            pl.BlockSpec((blk, _EMB), lambda i: (i, 0)),
            pl.BlockSpec((blk, 1), lambda i: (i, 0)),
            pl.BlockSpec((_EMB, _HID), lambda i: (0, 0)),
            pl.BlockSpec((_HID, _F), lambda i: (0, 0)),
        ],
        out_specs=pl.BlockSpec((blk, _F), lambda i: (i, 0)),
        out_shape=jax.ShapeDtypeStruct((_E, _F), jnp.float32),
    )(ee, ea, w0, w1)


def _sc_body(x_hbm, ew_hbm, src_hbm, dst_hbm, out_hbm,
             acc, src_idx, dst_idx, rows_v, ew_v, zbuf, sem_r, sem_w):
    c = lax.axis_index("c")
    s = lax.axis_index("s")
    wid = c * _NS + s

    def _zrow(r, carry):
        for cc in range(_F // 16):
            zbuf[r, pl.ds(cc * 16, 16)] = jnp.zeros((16,), jnp.float32)
        return carry
    lax.fori_loop(0, _WCH, _zrow, 0)
    for j in range(_NWCH):
        pltpu.sync_copy(zbuf, acc.at[pl.ds(s * _RPT + j * _WCH, _WCH)])
    plsc.subcore_barrier()

    def _chunk(i, carry):
        base = wid * _EPW + i * _CHUNK
        pltpu.sync_copy(src_hbm.at[pl.ds(base, _CHUNK)], src_idx)
        pltpu.sync_copy(dst_hbm.at[pl.ds(base, _CHUNK)], dst_idx)
        cp_ew = pltpu.async_copy(ew_hbm.at[pl.ds(base, _CHUNK)], ew_v, sem_w)
        cp_rows = pltpu.async_copy(x_hbm.at[src_idx], rows_v, sem_r)
        cp_ew.wait()
        cp_rows.wait()

        def _mrow(r, cr):
            for cc in range(_F // 16):
                sl = pl.ds(cc * 16, 16)
                rows_v[r, sl] = rows_v[r, sl] * ew_v[r, sl]
            return cr
        lax.fori_loop(0, _CHUNK, _mrow, 0)
        pltpu.sync_copy(rows_v, acc.at[dst_idx], add=True)
        return carry
    lax.fori_loop(0, _NCHUNK, _chunk, 0)

    plsc.subcore_barrier()
    for j in range(_NWCH):
        sl = pl.ds(s * _RPT + j * _WCH, _WCH)
        pltpu.sync_copy(acc.at[sl], zbuf)
        pltpu.sync_copy(zbuf, out_hbm.at[c].at[sl])


def _sc_scatter(x, ew, src, dst):
    mesh = plsc.VectorSubcoreMesh(core_axis_name="c", subcore_axis_name="s")
    f = pl.kernel(
        _sc_body,
        out_type=jax.ShapeDtypeStruct((_NC, _NPAD, _F), jnp.float32),
        mesh=mesh,
        scratch_types=[
            pltpu.VMEM_SHARED((_NPAD, _F), jnp.float32),
            pltpu.VMEM((_CHUNK,), jnp.int32),
            pltpu.VMEM((_CHUNK,), jnp.int32),
            pltpu.VMEM((_CHUNK, _F), jnp.float32),
            pltpu.VMEM((_CHUNK, _F), jnp.float32),
            pltpu.VMEM((_WCH, _F), jnp.float32),
            pltpu.SemaphoreType.DMA,
            pltpu.SemaphoreType.DMA,
        ],
    )
    return f(x, ew, src, dst)


def _final_body(p_ref, sc_ref, w2_ref, out_ref):
    ssum = p_ref[0] + p_ref[1]
    out_ref[...] = (
        jnp.dot(ssum, w2_ref[...], preferred_element_type=jnp.float32) + sc_ref[...]
    )


def _final_tc(partials, sc, w2s):
    blk = 1000
    grid = _N // blk
    return pl.pallas_call(
        _final_body,
        grid=(grid,),
        in_specs=[
            pl.BlockSpec((_NC, blk, _F), lambda i: (0, i, 0)),
            pl.BlockSpec((blk, _F), lambda i: (i, 0)),
            pl.BlockSpec((_F, _F), lambda i: (0, 0)),
        ],
        out_specs=pl.BlockSpec((blk, _F), lambda i: (i, 0)),
        out_shape=jax.ShapeDtypeStruct((_N, _F), jnp.float32),
    )(partials, sc, w2s)


def kernel(node_features, node_attrs, edge_src, edge_dst, edge_attr,
           edge_embedding, W1, mlp_w0, mlp_w1, W2, W_sc):
    w1s = W1 * np.float32(1.0 / np.sqrt(_F))
    wsc_t = jnp.transpose(W_sc, (1, 0, 2)) * np.float32(1.0 / np.sqrt(_F * _A))
    w2s = W2 * np.float32(1.0 / (np.sqrt(_F) * np.sqrt(32.0)))

    x, sc = _node_tc(node_features, node_attrs, w1s, wsc_t)
    ew = _edge_tc(edge_embedding, edge_attr, mlp_w0, mlp_w1)
    partials = _sc_scatter(x, ew, edge_src, edge_dst)
    return _final_tc(partials, sc, w2s)


# speedup vs baseline: 3.4628x; 1.4360x over previous
import functools

import numpy as np
import jax
import jax.numpy as jnp
from jax import lax
from jax.experimental import pallas as pl
from jax.experimental.pallas import tpu as pltpu
from jax.experimental.pallas import tpu_sc as plsc

_N = 10000
_E = 320000
_F = 128
_A = 4
_EMB = 16
_HID = 64

_NC = 2
_NS = 16
_NW = _NC * _NS
_EPW = _E // _NW
_CHUNK = 40
_NCHUNK = _EPW // _CHUNK
_NZCH = _N // _CHUNK


def _node_body(nf_ref, na_ref, w1_ref, wsc_ref, x_ref, sc_ref):
    nf = nf_ref[...]
    x_ref[...] = jnp.dot(nf, w1_ref[...], preferred_element_type=jnp.float32)
    acc = jnp.dot(nf, wsc_ref[0], preferred_element_type=jnp.float32) * na_ref[:, 0:1]
    for v in range(1, _A):
        acc = acc + jnp.dot(nf, wsc_ref[v], preferred_element_type=jnp.float32) * na_ref[:, v:v + 1]
    sc_ref[...] = acc


def _node_tc(nf, na, w1s, wsc_t):
    blk = 1000
    grid = _N // blk
    return pl.pallas_call(
        _node_body,
        grid=(grid,),
        in_specs=[
            pl.BlockSpec((blk, _F), lambda i: (i, 0)),
            pl.BlockSpec((blk, _A), lambda i: (i, 0)),
            pl.BlockSpec((_F, _F), lambda i: (0, 0)),
            pl.BlockSpec((_A, _F, _F), lambda i: (0, 0, 0)),
        ],
        out_specs=[
            pl.BlockSpec((blk, _F), lambda i: (i, 0)),
            pl.BlockSpec((blk, _F), lambda i: (i, 0)),
        ],
        out_shape=[
            jax.ShapeDtypeStruct((_N, _F), jnp.float32),
            jax.ShapeDtypeStruct((_N, _F), jnp.float32),
        ],
    )(nf, na, w1s, wsc_t)


def _edge_body(ee_ref, ea_ref, w0_ref, w1_ref, ew_ref):
    h = jnp.dot(ee_ref[...], w0_ref[...], preferred_element_type=jnp.float32)
    h = h * jax.nn.sigmoid(h)
    w = jnp.dot(h, w1_ref[...], preferred_element_type=jnp.float32)
    ew_ref[...] = w * ea_ref[...]


def _edge_tc(ee, ea, w0, w1):
    blk = 6400
    grid = _E // blk
    return pl.pallas_call(
        _edge_body,
        grid=(grid,),
        in_specs=[
            pl.BlockSpec((blk, _EMB), lambda i: (i, 0)),
            pl.BlockSpec((blk, 1), lambda i: (i, 0)),
            pl.BlockSpec((_EMB, _HID), lambda i: (0, 0)),
            pl.BlockSpec((_HID, _F), lambda i: (0, 0)),
        ],
        out_specs=pl.BlockSpec((blk, _F), lambda i: (i, 0)),
        out_shape=jax.ShapeDtypeStruct((_E, _F), jnp.float32),
    )(ee, ea, w0, w1)


def _sc_body(x_hbm, ew_hbm, src_hbm, dst_hbm, out_hbm,
             acc, src_all, dst_all, rows, ews, sem_r, sem_w, sem_s):
    c = lax.axis_index("c")
    s = lax.axis_index("s")
    wid = c * _NS + s

    pltpu.sync_copy(src_hbm.at[pl.ds(wid * _EPW, _EPW)], src_all)
    pltpu.sync_copy(dst_hbm.at[pl.ds(wid * _EPW, _EPW)], dst_all)

    def _zrow(r, carry):
        for cc in range(_F // 16):
            rows[0][r, pl.ds(cc * 16, 16)] = jnp.zeros((16,), jnp.float32)
        return carry
    lax.fori_loop(0, _CHUNK, _zrow, 0)
    for j in range(-(-_NZCH // _NS)):
        cid = s + _NS * j
        @pl.when(cid < _NZCH)
        def _():
            pltpu.sync_copy(rows[0], acc.at[pl.ds(cid * _CHUNK, _CHUNK)])
    plsc.subcore_barrier()

    def _issue(ci, sl):
        base = wid * _EPW + ci * _CHUNK
        pltpu.async_copy(ew_hbm.at[pl.ds(base, _CHUNK)], ews[sl], sem_w[sl])
        pltpu.async_copy(x_hbm.at[src_all.at[pl.ds(ci * _CHUNK, _CHUNK)]],
                         rows[sl], sem_r[sl])

    def _proc(ci, sl):
        pltpu.make_async_copy(ew_hbm.at[pl.ds(0, _CHUNK)], ews[sl], sem_w[sl]).wait()
        pltpu.make_async_copy(x_hbm.at[pl.ds(0, _CHUNK)], rows[sl], sem_r[sl]).wait()

        def _mrow(r, cr):
            for cc in range(_F // 16):
                g = pl.ds(cc * 16, 16)
                rows[sl][r, g] = rows[sl][r, g] * ews[sl][r, g]
            return cr
        lax.fori_loop(0, _CHUNK, _mrow, 0)
        pltpu.async_copy(rows[sl], acc.at[dst_all.at[pl.ds(ci * _CHUNK, _CHUNK)]],
                         sem_s[sl], add=True)

    def _wscat(sl):
        pltpu.make_async_copy(rows[sl], acc.at[pl.ds(0, _CHUNK)], sem_s[sl]).wait()

    _issue(0, 0)
    _issue(1, 1)
    _proc(0, 0)
    _issue(2, 2)
    _proc(1, 1)
    _wscat(0)
    _issue(3, 0)
    _proc(2, 2)

    def _triple(g, carry):
        ci = 3 * g
        _wscat(1)
        _issue(ci + 1, 1)
        _proc(ci, 0)
        _wscat(2)
        _issue(ci + 2, 2)
        _proc(ci + 1, 1)
        _wscat(0)
        _issue(ci + 3, 0)
        _proc(ci + 2, 2)
        return carry
    lax.fori_loop(1, 83, _triple, 0)

    _proc(249, 0)
    _wscat(1)
    _wscat(2)
    _wscat(0)

    plsc.subcore_barrier()
    for j in range(-(-_NZCH // _NS)):
        cid = s + _NS * j
        @pl.when(cid < _NZCH)
        def _():
            sl = pl.ds(cid * _CHUNK, _CHUNK)
            pltpu.sync_copy(acc.at[sl], rows[0])
            pltpu.sync_copy(rows[0], out_hbm.at[c].at[sl])


def _sc_scatter(x, ew, src3, dst3):
    mesh = plsc.VectorSubcoreMesh(core_axis_name="c", subcore_axis_name="s")
    f = pl.kernel(
        _sc_body,
        out_type=jax.ShapeDtypeStruct((_NC, _N, _F), jnp.float32),
        mesh=mesh,
        scratch_types=[
            pltpu.VMEM_SHARED((_N, _F), jnp.float32),
            pltpu.VMEM((_EPW,), jnp.int32),
            pltpu.VMEM((_EPW,), jnp.int32),
            [pltpu.VMEM((_CHUNK, _F), jnp.float32)] * 3,
            [pltpu.VMEM((_CHUNK, _F), jnp.float32)] * 3,
            [pltpu.SemaphoreType.DMA] * 3,
            [pltpu.SemaphoreType.DMA] * 3,
            [pltpu.SemaphoreType.DMA] * 3,
        ],
    )
    return f(x, ew, src3, dst3)


def _final_body(p_ref, sc_ref, w2_ref, out_ref):
    ssum = p_ref[0] + p_ref[1]
    out_ref[...] = (
        jnp.dot(ssum, w2_ref[...], preferred_element_type=jnp.float32) + sc_ref[...]
    )


def _final_tc(partials, sc, w2s):
    blk = 1000
    grid = _N // blk
    return pl.pallas_call(
        _final_body,
        grid=(grid,),
        in_specs=[
            pl.BlockSpec((_NC, blk, _F), lambda i: (0, i, 0)),
            pl.BlockSpec((blk, _F), lambda i: (i, 0)),
            pl.BlockSpec((_F, _F), lambda i: (0, 0)),
        ],
        out_specs=pl.BlockSpec((blk, _F), lambda i: (i, 0)),
        out_shape=jax.ShapeDtypeStruct((_N, _F), jnp.float32),
    )(partials, sc, w2s)


def kernel(node_features, node_attrs, edge_src, edge_dst, edge_attr,
           edge_embedding, W1, mlp_w0, mlp_w1, W2, W_sc):
    w1s = W1 * np.float32(1.0 / np.sqrt(_F))
    wsc_t = jnp.transpose(W_sc, (1, 0, 2)) * np.float32(1.0 / np.sqrt(_F * _A))
    w2s = W2 * np.float32(1.0 / (np.sqrt(_F) * np.sqrt(32.0)))

    x, sc = _node_tc(node_features, node_attrs, w1s, wsc_t)
    ew = _edge_tc(edge_embedding, edge_attr, mlp_w0, mlp_w1)
    partials = _sc_scatter(x, ew, edge_src, edge_dst)
    return _final_tc(partials, sc, w2s)


# speedup vs baseline: 5.5222x; 1.5947x over previous
import functools

import numpy as np
import jax
import jax.numpy as jnp
from jax import lax
from jax.experimental import pallas as pl
from jax.experimental.pallas import tpu as pltpu
from jax.experimental.pallas import tpu_sc as plsc

_N = 10000
_E = 320000
_F = 128
_A = 4
_EMB = 16
_HID = 64

_NC = 2
_NS = 16
_NW = _NC * _NS
_EPW = _E // _NW
_CHUNK = 40
_NCHUNK = _EPW // _CHUNK
_NZCH = _N // _CHUNK


def _node_body(nf_ref, na_ref, w1_ref, wsc_ref, x_ref, sc_ref):
    nf = nf_ref[...]
    x_ref[...] = jnp.dot(nf, w1_ref[...], preferred_element_type=jnp.float32)
    acc = jnp.dot(nf, wsc_ref[0], preferred_element_type=jnp.float32) * na_ref[:, 0:1]
    for v in range(1, _A):
        acc = acc + jnp.dot(nf, wsc_ref[v], preferred_element_type=jnp.float32) * na_ref[:, v:v + 1]
    sc_ref[...] = acc


def _node_tc(nf, na, w1s, wsc_t):
    blk = 1000
    grid = _N // blk
    return pl.pallas_call(
        _node_body,
        grid=(grid,),
        in_specs=[
            pl.BlockSpec((blk, _F), lambda i: (i, 0)),
            pl.BlockSpec((blk, _A), lambda i: (i, 0)),
            pl.BlockSpec((_F, _F), lambda i: (0, 0)),
            pl.BlockSpec((_A, _F, _F), lambda i: (0, 0, 0)),
        ],
        out_specs=[
            pl.BlockSpec((blk, _F), lambda i: (i, 0)),
            pl.BlockSpec((blk, _F), lambda i: (i, 0)),
        ],
        out_shape=[
            jax.ShapeDtypeStruct((_N, _F), jnp.float32),
            jax.ShapeDtypeStruct((_N, _F), jnp.float32),
        ],
    )(nf, na, w1s, wsc_t)


def _edge_body(eet_ref, w0_ref, w1_ref, ew_ref):
    h = lax.dot_general(eet_ref[...], w0_ref[...], (((0,), (0,)), ((), ())),
                        preferred_element_type=jnp.float32)
    h = h * jax.nn.sigmoid(h)
    ew_ref[...] = jnp.dot(h, w1_ref[...], preferred_element_type=jnp.float32)


def _edge_tc(eet, w0, w1):
    blk = 6400
    grid = _E // blk
    return pl.pallas_call(
        _edge_body,
        grid=(grid,),
        in_specs=[
            pl.BlockSpec((_EMB, blk), lambda i: (0, i)),
            pl.BlockSpec((_EMB, _HID), lambda i: (0, 0)),
            pl.BlockSpec((_HID, _F), lambda i: (0, 0)),
        ],
        out_specs=pl.BlockSpec((blk, _F), lambda i: (i, 0)),
        out_shape=jax.ShapeDtypeStruct((_E, _F), jnp.float32),
    )(eet, w0, w1)


def _sc_body(x_hbm, ew_hbm, ea_hbm, src_hbm, dst_hbm, out_hbm,
             acc, srcb, dstb, rows, ews, eas,
             sem_r, sem_w, sem_a, sem_s, sem_si, sem_di):
    c = lax.axis_index("c")
    s = lax.axis_index("s")
    wid = c * _NS + s

    def _zrow(r, carry):
        for cc in range(_F // 16):
            rows[0][r, pl.ds(cc * 16, 16)] = jnp.zeros((16,), jnp.float32)
        return carry
    lax.fori_loop(0, _CHUNK, _zrow, 0)
    for j in range(-(-_NZCH // _NS)):
        cid = s + _NS * j
        @pl.when(cid < _NZCH)
        def _():
            pltpu.sync_copy(rows[0], acc.at[pl.ds(cid * _CHUNK, _CHUNK)])
    plsc.subcore_barrier()

    def _isrc(ci, sl):
        base = wid * _EPW + ci * _CHUNK
        pltpu.async_copy(src_hbm.at[pl.ds(base, _CHUNK)], srcb[sl], sem_si[sl])

    def _idst(ci, sl):
        base = wid * _EPW + ci * _CHUNK
        pltpu.async_copy(dst_hbm.at[pl.ds(base, _CHUNK)], dstb[sl], sem_di[sl])

    def _ige(ci, sl):
        pltpu.make_async_copy(src_hbm.at[pl.ds(0, _CHUNK)], srcb[sl], sem_si[sl]).wait()
        base = wid * _EPW + ci * _CHUNK
        pltpu.async_copy(ew_hbm.at[pl.ds(base, _CHUNK)], ews[sl], sem_w[sl])
        pltpu.async_copy(ea_hbm.at[pl.ds(base, _CHUNK)],
                         eas[sl].at[pl.ds(0, _CHUNK)], sem_a[sl])
        pltpu.async_copy(x_hbm.at[srcb[sl]], rows[sl], sem_r[sl])

    def _proc(ci, sl):
        pltpu.make_async_copy(ew_hbm.at[pl.ds(0, _CHUNK)], ews[sl], sem_w[sl]).wait()
        pltpu.make_async_copy(ea_hbm.at[pl.ds(0, _CHUNK)],
                              eas[sl].at[pl.ds(0, _CHUNK)], sem_a[sl]).wait()
        pltpu.make_async_copy(x_hbm.at[pl.ds(0, _CHUNK)], rows[sl], sem_r[sl]).wait()
        pltpu.make_async_copy(dst_hbm.at[pl.ds(0, _CHUNK)], dstb[sl], sem_di[sl]).wait()

        def _mrow(r, cr):
            av = eas[sl][pl.ds(r, 16)][0]
            for cc in range(_F // 16):
                g = pl.ds(cc * 16, 16)
                rows[sl][r, g] = rows[sl][r, g] * ews[sl][r, g] * av
            return cr
        lax.fori_loop(0, _CHUNK, _mrow, 0)
        pltpu.async_copy(rows[sl], acc.at[dstb[sl]], sem_s[sl], add=True)

    def _wscat(sl):
        pltpu.make_async_copy(rows[sl], acc.at[pl.ds(0, _CHUNK)], sem_s[sl]).wait()

    def _B(ci, ph, wsc):
        s0, s1, s2 = ph, (ph + 1) % 3, (ph + 2) % 3
        if wsc:
            _wscat(s1)
        _idst(ci + 1, s1)
        _ige(ci + 1, s1)
        _isrc(ci + 2, s2)
        _proc(ci, s0)

    _isrc(0, 0)
    _idst(0, 0)
    _isrc(1, 1)
    _ige(0, 0)
    _B(0, 0, wsc=False)
    _B(1, 1, wsc=False)
    _B(2, 2, wsc=True)

    def _triple(g, carry):
        ci = 3 * g
        _B(ci, 0, wsc=True)
        _B(ci + 1, 1, wsc=True)
        _B(ci + 2, 2, wsc=True)
        return carry
    lax.fori_loop(1, 82, _triple, 0)

    _wscat(1)
    _idst(247, 1)
    _ige(247, 1)
    _isrc(248, 2)
    _proc(246, 0)
    _wscat(2)
    _idst(248, 2)
    _ige(248, 2)
    _isrc(249, 0)
    _proc(247, 1)
    _wscat(0)
    _idst(249, 0)
    _ige(249, 0)
    _proc(248, 2)
    _wscat(1)
    _proc(249, 0)
    _wscat(2)
    _wscat(0)

    plsc.subcore_barrier()
    for j in range(-(-_NZCH // _NS)):
        cid = s + _NS * j
        @pl.when(cid < _NZCH)
        def _():
            sl = pl.ds(cid * _CHUNK, _CHUNK)
            pltpu.sync_copy(acc.at[sl], rows[0])
            pltpu.sync_copy(rows[0], out_hbm.at[c].at[sl])


def _sc_scatter(x, ew, ea, src, dst):
    mesh = plsc.VectorSubcoreMesh(core_axis_name="c", subcore_axis_name="s")
    f = pl.kernel(
        _sc_body,
        out_type=jax.ShapeDtypeStruct((_NC, _N, _F), jnp.float32),
        mesh=mesh,
        scratch_types=[
            pltpu.VMEM_SHARED((_N, _F), jnp.float32),
            [pltpu.VMEM((_CHUNK,), jnp.int32)] * 3,
            [pltpu.VMEM((_CHUNK,), jnp.int32)] * 3,
            [pltpu.VMEM((_CHUNK, _F), jnp.float32)] * 3,
            [pltpu.VMEM((_CHUNK, _F), jnp.float32)] * 3,
            [pltpu.VMEM((_CHUNK + 16,), jnp.float32)] * 3,
            [pltpu.SemaphoreType.DMA] * 3,
            [pltpu.SemaphoreType.DMA] * 3,
            [pltpu.SemaphoreType.DMA] * 3,
            [pltpu.SemaphoreType.DMA] * 3,
            [pltpu.SemaphoreType.DMA] * 3,
            [pltpu.SemaphoreType.DMA] * 3,
        ],
    )
    return f(x, ew, ea, src, dst)


def _final_body(p_ref, sc_ref, w2_ref, out_ref):
    ssum = p_ref[0] + p_ref[1]
    out_ref[...] = (
        jnp.dot(ssum, w2_ref[...], preferred_element_type=jnp.float32) + sc_ref[...]
    )


def _final_tc(partials, sc, w2s):
    blk = 1000
    grid = _N // blk
    return pl.pallas_call(
        _final_body,
        grid=(grid,),
        in_specs=[
            pl.BlockSpec((_NC, blk, _F), lambda i: (0, i, 0)),
            pl.BlockSpec((blk, _F), lambda i: (i, 0)),
            pl.BlockSpec((_F, _F), lambda i: (0, 0)),
        ],
        out_specs=pl.BlockSpec((blk, _F), lambda i: (i, 0)),
        out_shape=jax.ShapeDtypeStruct((_N, _F), jnp.float32),
    )(partials, sc, w2s)


def kernel(node_features, node_attrs, edge_src, edge_dst, edge_attr,
           edge_embedding, W1, mlp_w0, mlp_w1, W2, W_sc):
    w1s = W1 * np.float32(1.0 / np.sqrt(_F))
    wsc_t = jnp.transpose(W_sc, (1, 0, 2)) * np.float32(1.0 / np.sqrt(_F * _A))
    w2s = W2 * np.float32(1.0 / (np.sqrt(_F) * np.sqrt(32.0)))

    x, sc = _node_tc(node_features, node_attrs, w1s, wsc_t)
    ew = _edge_tc(edge_embedding.T, mlp_w0, mlp_w1)
    partials = _sc_scatter(x, ew, edge_attr.reshape(_E), edge_src, edge_dst)
    return _final_tc(partials, sc, w2s)
